# Initial kernel scaffold; baseline (speedup 1.0000x reference)
import jax
import jax.numpy as jnp
from jax.experimental import pallas as pl


def kernel(x, params, neigh_orders):
    raise NotImplementedError("write your pallas kernel here")


# speedup vs baseline: 2.5456x; 2.5456x over previous
import functools

import numpy as np

import jax
import jax.numpy as jnp
from jax import lax
from jax.experimental import pallas as pl
from jax.experimental.pallas import tpu as pltpu
from jax.experimental.pallas import tpu_sc as plsc

NW = 32
NT = 16

_LEVELS = [10242, 2562, 642, 162, 42]
_CONVS = [
    ("1_1", 3, 64, 10242), ("1_2", 64, 64, 10242),
    ("2_1", 64, 128, 2562), ("2_2", 128, 128, 2562),
    ("3_1", 128, 256, 642), ("3_2", 256, 256, 642),
    ("4_1", 256, 512, 162), ("4_2", 512, 512, 162),
    ("5_1", 512, 1024, 42), ("5_2", 1024, 512, 42),
    ("6_1", 512, 512, 162), ("6_2", 512, 256, 162),
    ("7_1", 256, 256, 642), ("7_2", 256, 128, 642),
    ("8_1", 128, 128, 2562), ("8_2", 128, 64, 2562),
    ("9_1", 64, 64, 10242), ("9_2", 64, 64, 10242),
]


def _rup(a, m):
    return (a + m - 1) // m * m


def _gather_g(d):
    if d <= 256:
        return 128
    if d <= 512:
        return 64
    return 32


def _sc_gather(table, idx, b):
    d = table.shape[1]
    g_sz = _gather_g(d)
    ng = b // g_sz
    t_rounds = -(-ng // NW)
    ngp = t_rounds * NW
    idxp = jnp.pad(idx, (0, ngp * g_sz - b))
    idx3 = idxp.reshape(t_rounds, NW, g_sz).transpose(1, 0, 2)

    mesh = plsc.VectorSubcoreMesh(core_axis_name="c", subcore_axis_name="s")

    def body(table_hbm, idx_hbm, out_hbm, idx_v, rows_v, sem):
        w = lax.axis_index("s") * 2 + lax.axis_index("c")
        pltpu.sync_copy(idx_hbm.at[w], idx_v)
        for t in range(t_rounds):
            grp = t * NW + w

            @pl.when(grp < ng)
            def _():
                pltpu.async_copy(table_hbm.at[idx_v.at[t]], rows_v, sem).wait()
                pltpu.sync_copy(rows_v, out_hbm.at[pl.ds(grp * g_sz, g_sz)])

    return pl.kernel(
        body,
        out_type=jax.ShapeDtypeStruct((b, d), jnp.float32),
        mesh=mesh,
        compiler_params=pltpu.CompilerParams(use_tc_tiling_on_sc=False),
        scratch_types=[
            pltpu.VMEM((t_rounds, g_sz), jnp.int32),
            pltpu.VMEM((g_sz, d), jnp.float32),
            pltpu.SemaphoreType.DMA,
        ],
    )(table, idx3)


def _kmaj_idx(no, n, n_pad):
    idx = no[: n * 7].reshape(n, 7).T
    return jnp.pad(idx, ((0, 0), (0, n_pad - n))).reshape(-1)


def _gather7(table, no, n, n_pad):
    idx = jnp.pad(no[: n * 7], (0, 7 * (n_pad - n)))
    flat = _sc_gather(table, idx, 7 * n_pad)
    return flat.reshape(n_pad, 7 * table.shape[1])


def _tc_conv(gath, wt, b):
    n_pad, kdim = gath.shape
    m = wt.shape[1]
    rb = n_pad if n_pad <= 2688 else 2592
    grid = n_pad // rb

    def body(g_ref, w_ref, b_ref, o_ref):
        o_ref[...] = jnp.dot(g_ref[...], w_ref[...],
                             preferred_element_type=jnp.float32) + b_ref[...]

    return pl.pallas_call(
        body,
        grid=(grid,),
        in_specs=[
            pl.BlockSpec((rb, kdim), lambda i: (i, 0)),
            pl.BlockSpec((kdim, m), lambda i: (0, 0)),
            pl.BlockSpec((1, m), lambda i: (0, 0)),
        ],
        out_specs=pl.BlockSpec((rb, m), lambda i: (i, 0)),
        out_shape=jax.ShapeDtypeStruct((n_pad, m), jnp.float32),
    )(gath, wt, b)


def _tc_bnrelu(h, g, be, n):
    n_pad, m = h.shape
    rcp = float(np.float32(1.0) / np.float32(n))
    wide = n == 10242
    rows = 128 if wide else 8

    def body(h_ref, g_ref, b_ref, o_ref):
        def tile_sum(transform):
            def step(i, acc):
                blk = transform(h_ref[pl.ds(i * rows, rows), :])
                rid = lax.broadcasted_iota(jnp.int32, (rows, m), 0) + i * rows
                return acc + jnp.where(rid < n, blk, 0.0)

            acc = lax.fori_loop(0, n_pad // rows, step,
                                jnp.zeros((rows, m), jnp.float32))
            if wide:
                p = acc[0:8]
                for j in range(1, 16):
                    p = p + acc[8 * j:8 * j + 8]
            else:
                p = acc
            q = p[0:4] + p[4:8]
            q = q[0:2] + q[2:4]
            return q[0:1] + q[1:2]

        mu = tile_sum(lambda b: b) * rcp
        var = tile_sum(lambda b: (b - mu) * (b - mu)) * rcp
        y = (h_ref[...] - mu) / jnp.sqrt(var + 1e-5) * g_ref[...] + b_ref[...]
        o_ref[...] = jnp.maximum(y, 0.0)

    return pl.pallas_call(
        body,
        out_shape=jax.ShapeDtypeStruct((n_pad, m), jnp.float32),
    )(h, g.reshape(1, m), be.reshape(1, m))


def _tc_pool(gath):
    _, num_pad, c = gath.shape

    def body(g_ref, mx_ref, mi_ref):
        best = g_ref[0]
        bi = jnp.zeros((num_pad, c), jnp.int32)
        for k in range(1, 7):
            cur = g_ref[k]
            upd = cur > best
            best = jnp.where(upd, cur, best)
            bi = jnp.where(upd, k, bi)
        mx_ref[...] = best
        mi_ref[...] = bi

    return pl.pallas_call(
        body,
        out_shape=(
            jax.ShapeDtypeStruct((num_pad, c), jnp.float32),
            jax.ShapeDtypeStruct((num_pad, c), jnp.int32),
        ),
    )(gath)


def _tc_unpool_vals(h, mi, num):
    num_pad, f = h.shape

    def body(h_ref, mi_ref, o_ref):
        mask = lax.broadcasted_iota(jnp.int32, (num_pad, f), 0) < num
        xv = jnp.where(mask, h_ref[...], 0.0)
        mv = mi_ref[...]
        for k in range(7):
            o_ref[k] = jnp.where(mv == k, xv, 0.0)

    return pl.pallas_call(
        body,
        out_shape=jax.ShapeDtypeStruct((7, num_pad, f), jnp.float32),
    )(h, mi)


def _sc_unpool(vals, nb4, ro):
    _, np_, f = vals.shape
    ngs = np_ // 32
    t_rounds = -(-ngs // NT)
    rows_pt = ro // NT
    zeros = jnp.zeros((ro, f), jnp.float32)

    mesh = plsc.VectorSubcoreMesh(core_axis_name="c", subcore_axis_name="s")

    def body(vals_hbm, nb_hbm, z_hbm, out_hbm, val_v, nb_v, shared):
        cid = lax.axis_index("c")
        sid = lax.axis_index("s")

        @pl.when(cid == 0)
        def _():
            pltpu.sync_copy(z_hbm.at[pl.ds(sid * rows_pt, rows_pt)],
                            shared.at[pl.ds(sid * rows_pt, rows_pt)])

        plsc.subcore_barrier()

        @pl.when(cid == 0)
        def _():
            for t in range(t_rounds):
                grp = t * NT + sid

                @pl.when(grp < ngs)
                def _():
                    pltpu.sync_copy(nb_hbm.at[grp], nb_v)
                    for k in range(7):
                        pltpu.sync_copy(vals_hbm.at[k, pl.ds(grp * 32, 32)], val_v)
                        pltpu.sync_copy(val_v, shared.at[nb_v.at[k]], add=True)

        plsc.subcore_barrier()

        @pl.when(cid == 0)
        def _():
            pltpu.sync_copy(shared.at[pl.ds(sid * rows_pt, rows_pt)],
                            out_hbm.at[pl.ds(sid * rows_pt, rows_pt)])

    return pl.kernel(
        body,
        out_type=jax.ShapeDtypeStruct((ro, f), jnp.float32),
        mesh=mesh,
        compiler_params=pltpu.CompilerParams(use_tc_tiling_on_sc=False),
        scratch_types=[
            pltpu.VMEM((32, f), jnp.float32),
            pltpu.VMEM((7, 32), jnp.int32),
            pltpu.VMEM_SHARED((ro, f), jnp.float32),
        ],
    )(vals, nb4, zeros)


def _unpool(h, mi, no, num, ro):
    vals = _tc_unpool_vals(h, mi, num)
    np_ = h.shape[0]
    nb = no[: num * 7].reshape(num, 7).T
    nb = jnp.pad(nb, ((0, 0), (0, np_ - num)))
    nb4 = nb.reshape(7, np_ // 32, 32).transpose(1, 0, 2)
    return _sc_unpool(vals, nb4, ro)


def _prep_w(w, cin, cout, d_pad, m_pad):
    wt = w.reshape(cout, 7, cin)
    wt = jnp.pad(wt, ((0, m_pad - cout), (0, 0), (0, d_pad - cin)))
    return wt.reshape(m_pad, 7 * d_pad).T


def kernel(x, params, neigh_orders):
    no = neigh_orders
    npad = {n: _rup(n, 128) for n in _LEVELS}

    def cbr(h, name, n):
        cin = params["W_" + name].shape[1] // 7
        cout = params["W_" + name].shape[0]
        gath = _gather7(h, no[str(n)], n, npad[n])
        if cin != h.shape[1]:
            gath = gath.reshape(npad[n], 7, h.shape[1])[:, :, :cin]
            gath = gath.reshape(npad[n], 7 * cin)
        wt = _prep_w(params["W_" + name], cin, cout, cin, cout)
        hh = _tc_conv(gath, wt, params["b_" + name].reshape(1, cout))
        return _tc_bnrelu(hh, params["g_" + name], params["be_" + name], n)

    def pool(h, n):
        num = (n + 6) // 4
        idx = _kmaj_idx(no[str(n)], num, npad[num])
        gath = _sc_gather(h, idx, 7 * npad[num]).reshape(7, npad[num], h.shape[1])
        return _tc_pool(gath)

    h = jnp.pad(x, ((0, 0), (0, 13)))
    h = cbr(h, "1_1", 10242)
    h = cbr(h, "1_2", 10242)
    h, mi1 = pool(h, 10242)
    h = cbr(h, "2_1", 2562)
    h = cbr(h, "2_2", 2562)
    h, mi2 = pool(h, 2562)
    h = cbr(h, "3_1", 642)
    h = cbr(h, "3_2", 642)
    h, mi3 = pool(h, 642)
    h = cbr(h, "4_1", 162)
    h = cbr(h, "4_2", 162)
    h, mi4 = pool(h, 162)
    h = cbr(h, "5_1", 42)
    h = cbr(h, "5_2", 42)
    h = _unpool(h, mi4, no["162"], 42, _rup(162, 16 * 32))
    h = cbr(h, "6_1", 162)
    h = cbr(h, "6_2", 162)
    h = _unpool(h, mi3, no["642"], 162, _rup(642, 16 * 32))
    h = cbr(h, "7_1", 642)
    h = cbr(h, "7_2", 642)
    h = _unpool(h, mi2, no["2562"], 642, _rup(2562, 16 * 32))
    h = cbr(h, "8_1", 2562)
    h = cbr(h, "8_2", 2562)
    h = _unpool(h, mi1, no["10242"], 2562, _rup(10242, 16 * 32))
    h = cbr(h, "9_1", 10242)
    h = cbr(h, "9_2", 10242)
    wt = _prep_w(params["W_10"], 64, 36, 64, 128)
    gath = _gather7(h, no["10242"], 10242, npad[10242])
    b10 = jnp.pad(params["b_10"], (0, 128 - 36)).reshape(1, 128)
    out = _tc_conv(gath, wt, b10)
    return out[:10242, :36]


# speedup vs baseline: 2.6059x; 1.0237x over previous
import functools

import numpy as np

import jax
import jax.numpy as jnp
from jax import lax
from jax.experimental import pallas as pl
from jax.experimental.pallas import tpu as pltpu
from jax.experimental.pallas import tpu_sc as plsc

NW = 32
NT = 16

_LEVELS = [10242, 2562, 642, 162, 42]
_CONVS = [
    ("1_1", 3, 64, 10242), ("1_2", 64, 64, 10242),
    ("2_1", 64, 128, 2562), ("2_2", 128, 128, 2562),
    ("3_1", 128, 256, 642), ("3_2", 256, 256, 642),
    ("4_1", 256, 512, 162), ("4_2", 512, 512, 162),
    ("5_1", 512, 1024, 42), ("5_2", 1024, 512, 42),
    ("6_1", 512, 512, 162), ("6_2", 512, 256, 162),
    ("7_1", 256, 256, 642), ("7_2", 256, 128, 642),
    ("8_1", 128, 128, 2562), ("8_2", 128, 64, 2562),
    ("9_1", 64, 64, 10242), ("9_2", 64, 64, 10242),
]


def _rup(a, m):
    return (a + m - 1) // m * m


def _gather_g(d):
    if d <= 256:
        return 128
    if d <= 512:
        return 64
    return 32


def _sc_gather(table, idx, b):
    d = table.shape[1]
    g_sz = _gather_g(d)
    ng = b // g_sz
    t_rounds = -(-ng // NW)
    ngp = t_rounds * NW
    idxp = jnp.pad(idx, (0, ngp * g_sz - b))
    idx3 = idxp.reshape(t_rounds, NW, g_sz).transpose(1, 0, 2)

    mesh = plsc.VectorSubcoreMesh(core_axis_name="c", subcore_axis_name="s")

    def body(table_hbm, idx_hbm, out_hbm, idx_v, buf0, buf1, gs0, gs1, ws0, ws1):
        w = lax.axis_index("s") * 2 + lax.axis_index("c")
        pltpu.sync_copy(idx_hbm.at[w], idx_v)
        bufs, gsems, wsems = (buf0, buf1), (gs0, gs1), (ws0, ws1)
        last_ok = w < ng - (t_rounds - 1) * NW

        def guarded(t, fn):
            if t == t_rounds - 1:
                @pl.when(last_ok)
                def _():
                    fn()
            else:
                fn()

        gh = [pltpu.make_async_copy(table_hbm.at[idx_v.at[t]],
                                    bufs[t % 2], gsems[t % 2])
              for t in range(t_rounds)]
        wh = [pltpu.make_async_copy(bufs[t % 2],
                                    out_hbm.at[pl.ds((t * NW + w) * g_sz, g_sz)],
                                    wsems[t % 2])
              for t in range(t_rounds)]
        guarded(0, gh[0].start)
        for t in range(t_rounds):
            if t + 1 < t_rounds:
                if t - 1 >= 0:
                    guarded(t - 1, wh[t - 1].wait)
                guarded(t + 1, gh[t + 1].start)
            guarded(t, gh[t].wait)
            guarded(t, wh[t].start)
        for t in (t_rounds - 2, t_rounds - 1):
            if t >= 0:
                guarded(t, wh[t].wait)

    return pl.kernel(
        body,
        out_type=jax.ShapeDtypeStruct((b, d), jnp.float32),
        mesh=mesh,
        compiler_params=pltpu.CompilerParams(use_tc_tiling_on_sc=False,
                                             skip_device_barrier=True),
        scratch_types=[
            pltpu.VMEM((t_rounds, g_sz), jnp.int32),
            pltpu.VMEM((g_sz, d), jnp.float32),
            pltpu.VMEM((g_sz, d), jnp.float32),
            pltpu.SemaphoreType.DMA,
            pltpu.SemaphoreType.DMA,
            pltpu.SemaphoreType.DMA,
            pltpu.SemaphoreType.DMA,
        ],
    )(table, idx3)


def _kmaj_idx(no, n, n_pad):
    idx = no[: n * 7].reshape(n, 7).T
    return jnp.pad(idx, ((0, 0), (0, n_pad - n))).reshape(-1)


def _gather7(table, no, n, n_pad):
    idx = jnp.pad(no[: n * 7], (0, 7 * (n_pad - n)))
    flat = _sc_gather(table, idx, 7 * n_pad)
    return flat.reshape(n_pad, 7 * table.shape[1])


def _tc_conv(gath, wt, b):
    n_pad, kdim = gath.shape
    m = wt.shape[1]
    rb = n_pad if n_pad <= 2688 else 2592
    grid = n_pad // rb

    def body(g_ref, w_ref, b_ref, o_ref):
        o_ref[...] = jnp.dot(g_ref[...], w_ref[...],
                             preferred_element_type=jnp.float32) + b_ref[...]

    return pl.pallas_call(
        body,
        grid=(grid,),
        in_specs=[
            pl.BlockSpec((rb, kdim), lambda i: (i, 0)),
            pl.BlockSpec((kdim, m), lambda i: (0, 0)),
            pl.BlockSpec((1, m), lambda i: (0, 0)),
        ],
        out_specs=pl.BlockSpec((rb, m), lambda i: (i, 0)),
        out_shape=jax.ShapeDtypeStruct((n_pad, m), jnp.float32),
    )(gath, wt, b)


def _tc_bnrelu(h, g, be, n):
    n_pad, m = h.shape
    rcp = float(np.float32(1.0) / np.float32(n))
    wide = n == 10242
    rows = 128 if wide else 8

    def body(h_ref, g_ref, b_ref, o_ref):
        def tile_sum(transform):
            def step(i, acc):
                blk = transform(h_ref[pl.ds(i * rows, rows), :])
                rid = lax.broadcasted_iota(jnp.int32, (rows, m), 0) + i * rows
                return acc + jnp.where(rid < n, blk, 0.0)

            acc = lax.fori_loop(0, n_pad // rows, step,
                                jnp.zeros((rows, m), jnp.float32))
            if wide:
                p = acc[0:8]
                for j in range(1, 16):
                    p = p + acc[8 * j:8 * j + 8]
            else:
                p = acc
            q = p[0:4] + p[4:8]
            q = q[0:2] + q[2:4]
            return q[0:1] + q[1:2]

        mu = tile_sum(lambda b: b) * rcp
        var = tile_sum(lambda b: (b - mu) * (b - mu)) * rcp
        y = (h_ref[...] - mu) / jnp.sqrt(var + 1e-5) * g_ref[...] + b_ref[...]
        o_ref[...] = jnp.maximum(y, 0.0)

    return pl.pallas_call(
        body,
        out_shape=jax.ShapeDtypeStruct((n_pad, m), jnp.float32),
    )(h, g.reshape(1, m), be.reshape(1, m))


def _tc_pool(gath):
    _, num_pad, c = gath.shape

    def body(g_ref, mx_ref, mi_ref):
        best = g_ref[0]
        bi = jnp.zeros((num_pad, c), jnp.int32)
        for k in range(1, 7):
            cur = g_ref[k]
            upd = cur > best
            best = jnp.where(upd, cur, best)
            bi = jnp.where(upd, k, bi)
        mx_ref[...] = best
        mi_ref[...] = bi

    return pl.pallas_call(
        body,
        out_shape=(
            jax.ShapeDtypeStruct((num_pad, c), jnp.float32),
            jax.ShapeDtypeStruct((num_pad, c), jnp.int32),
        ),
    )(gath)


def _tc_unpool_vals(h, mi, num):
    num_pad, f = h.shape

    def body(h_ref, mi_ref, o_ref):
        mask = lax.broadcasted_iota(jnp.int32, (num_pad, f), 0) < num
        xv = jnp.where(mask, h_ref[...], 0.0)
        mv = mi_ref[...]
        for k in range(7):
            o_ref[k] = jnp.where(mv == k, xv, 0.0)

    return pl.pallas_call(
        body,
        out_shape=jax.ShapeDtypeStruct((7, num_pad, f), jnp.float32),
    )(h, mi)


def _sc_unpool(vals, nb4, ro):
    _, np_, f = vals.shape
    ngs = np_ // 32
    t_rounds = -(-ngs // NT)
    rows_pt = ro // NT
    zeros = jnp.zeros((ro, f), jnp.float32)

    mesh = plsc.VectorSubcoreMesh(core_axis_name="c", subcore_axis_name="s")

    def body(vals_hbm, nb_hbm, z_hbm, out_hbm, val_v, nb_v, shared):
        cid = lax.axis_index("c")
        sid = lax.axis_index("s")

        @pl.when(cid == 0)
        def _():
            pltpu.sync_copy(z_hbm.at[pl.ds(sid * rows_pt, rows_pt)],
                            shared.at[pl.ds(sid * rows_pt, rows_pt)])

        plsc.subcore_barrier()

        @pl.when(cid == 0)
        def _():
            for t in range(t_rounds):
                grp = t * NT + sid

                @pl.when(grp < ngs)
                def _():
                    pltpu.sync_copy(nb_hbm.at[grp], nb_v)
                    for k in range(7):
                        pltpu.sync_copy(vals_hbm.at[k, pl.ds(grp * 32, 32)], val_v)
                        pltpu.sync_copy(val_v, shared.at[nb_v.at[k]], add=True)

        plsc.subcore_barrier()

        @pl.when(cid == 0)
        def _():
            pltpu.sync_copy(shared.at[pl.ds(sid * rows_pt, rows_pt)],
                            out_hbm.at[pl.ds(sid * rows_pt, rows_pt)])

    return pl.kernel(
        body,
        out_type=jax.ShapeDtypeStruct((ro, f), jnp.float32),
        mesh=mesh,
        compiler_params=pltpu.CompilerParams(use_tc_tiling_on_sc=False,
                                             skip_device_barrier=True),
        scratch_types=[
            pltpu.VMEM((32, f), jnp.float32),
            pltpu.VMEM((7, 32), jnp.int32),
            pltpu.VMEM_SHARED((ro, f), jnp.float32),
        ],
    )(vals, nb4, zeros)


def _unpool(h, mi, no, num, ro):
    vals = _tc_unpool_vals(h, mi, num)
    np_ = h.shape[0]
    nb = no[: num * 7].reshape(num, 7).T
    nb = jnp.pad(nb, ((0, 0), (0, np_ - num)))
    nb4 = nb.reshape(7, np_ // 32, 32).transpose(1, 0, 2)
    return _sc_unpool(vals, nb4, ro)


def _prep_w(w, cin, cout, d_pad, m_pad):
    wt = w.reshape(cout, 7, cin)
    wt = jnp.pad(wt, ((0, m_pad - cout), (0, 0), (0, d_pad - cin)))
    return wt.reshape(m_pad, 7 * d_pad).T


def kernel(x, params, neigh_orders):
    no = neigh_orders
    npad = {n: _rup(n, 128) for n in _LEVELS}

    def cbr(h, name, n):
        cin = params["W_" + name].shape[1] // 7
        cout = params["W_" + name].shape[0]
        gath = _gather7(h, no[str(n)], n, npad[n])
        if cin != h.shape[1]:
            gath = gath.reshape(npad[n], 7, h.shape[1])[:, :, :cin]
            gath = gath.reshape(npad[n], 7 * cin)
        wt = _prep_w(params["W_" + name], cin, cout, cin, cout)
        hh = _tc_conv(gath, wt, params["b_" + name].reshape(1, cout))
        return _tc_bnrelu(hh, params["g_" + name], params["be_" + name], n)

    def pool(h, n):
        num = (n + 6) // 4
        idx = _kmaj_idx(no[str(n)], num, npad[num])
        gath = _sc_gather(h, idx, 7 * npad[num]).reshape(7, npad[num], h.shape[1])
        return _tc_pool(gath)

    h = jnp.pad(x, ((0, 0), (0, 13)))
    h = cbr(h, "1_1", 10242)
    h = cbr(h, "1_2", 10242)
    h, mi1 = pool(h, 10242)
    h = cbr(h, "2_1", 2562)
    h = cbr(h, "2_2", 2562)
    h, mi2 = pool(h, 2562)
    h = cbr(h, "3_1", 642)
    h = cbr(h, "3_2", 642)
    h, mi3 = pool(h, 642)
    h = cbr(h, "4_1", 162)
    h = cbr(h, "4_2", 162)
    h, mi4 = pool(h, 162)
    h = cbr(h, "5_1", 42)
    h = cbr(h, "5_2", 42)
    h = _unpool(h, mi4, no["162"], 42, _rup(162, 16 * 32))
    h = cbr(h, "6_1", 162)
    h = cbr(h, "6_2", 162)
    h = _unpool(h, mi3, no["642"], 162, _rup(642, 16 * 32))
    h = cbr(h, "7_1", 642)
    h = cbr(h, "7_2", 642)
    h = _unpool(h, mi2, no["2562"], 642, _rup(2562, 16 * 32))
    h = cbr(h, "8_1", 2562)
    h = cbr(h, "8_2", 2562)
    h = _unpool(h, mi1, no["10242"], 2562, _rup(10242, 16 * 32))
    h = cbr(h, "9_1", 10242)
    h = cbr(h, "9_2", 10242)
    wt = _prep_w(params["W_10"], 64, 36, 64, 128)
    gath = _gather7(h, no["10242"], 10242, npad[10242])
    b10 = jnp.pad(params["b_10"], (0, 128 - 36)).reshape(1, 128)
    out = _tc_conv(gath, wt, b10)
    return out[:10242, :36]


# speedup vs baseline: 2.6781x; 1.0277x over previous
import functools

import numpy as np

import jax
import jax.numpy as jnp
from jax import lax
from jax.experimental import pallas as pl
from jax.experimental.pallas import tpu as pltpu
from jax.experimental.pallas import tpu_sc as plsc

NW = 32
NT = 16

_LEVELS = [10242, 2562, 642, 162, 42]
_CONVS = [
    ("1_1", 3, 64, 10242), ("1_2", 64, 64, 10242),
    ("2_1", 64, 128, 2562), ("2_2", 128, 128, 2562),
    ("3_1", 128, 256, 642), ("3_2", 256, 256, 642),
    ("4_1", 256, 512, 162), ("4_2", 512, 512, 162),
    ("5_1", 512, 1024, 42), ("5_2", 1024, 512, 42),
    ("6_1", 512, 512, 162), ("6_2", 512, 256, 162),
    ("7_1", 256, 256, 642), ("7_2", 256, 128, 642),
    ("8_1", 128, 128, 2562), ("8_2", 128, 64, 2562),
    ("9_1", 64, 64, 10242), ("9_2", 64, 64, 10242),
]


def _rup(a, m):
    return (a + m - 1) // m * m


def _gather_g(d):
    if d <= 256:
        return 128
    if d <= 512:
        return 64
    return 32


def _sc_gather(table, idx, b):
    d = table.shape[1]
    g_sz = _gather_g(d)
    ng = b // g_sz
    t_rounds = -(-ng // NW)
    ngp = t_rounds * NW
    idxp = jnp.pad(idx, (0, ngp * g_sz - b))
    idx3 = idxp.reshape(t_rounds, NW, g_sz).transpose(1, 0, 2)

    mesh = plsc.VectorSubcoreMesh(core_axis_name="c", subcore_axis_name="s")

    def body(table_hbm, idx_hbm, out_hbm, idx_v, buf0, buf1, gs0, gs1, ws0, ws1):
        w = lax.axis_index("s") * 2 + lax.axis_index("c")
        pltpu.sync_copy(idx_hbm.at[w], idx_v)
        bufs, gsems, wsems = (buf0, buf1), (gs0, gs1), (ws0, ws1)
        last_ok = w < ng - (t_rounds - 1) * NW

        def guarded(t, fn):
            if t == t_rounds - 1:
                @pl.when(last_ok)
                def _():
                    fn()
            else:
                fn()

        gh = [pltpu.make_async_copy(table_hbm.at[idx_v.at[t]],
                                    bufs[t % 2], gsems[t % 2])
              for t in range(t_rounds)]
        wh = [pltpu.make_async_copy(bufs[t % 2],
                                    out_hbm.at[pl.ds((t * NW + w) * g_sz, g_sz)],
                                    wsems[t % 2])
              for t in range(t_rounds)]
        guarded(0, gh[0].start)
        for t in range(t_rounds):
            if t + 1 < t_rounds:
                if t - 1 >= 0:
                    guarded(t - 1, wh[t - 1].wait)
                guarded(t + 1, gh[t + 1].start)
            guarded(t, gh[t].wait)
            guarded(t, wh[t].start)
        for t in (t_rounds - 2, t_rounds - 1):
            if t >= 0:
                guarded(t, wh[t].wait)

    return pl.kernel(
        body,
        out_type=jax.ShapeDtypeStruct((b, d), jnp.float32),
        mesh=mesh,
        compiler_params=pltpu.CompilerParams(use_tc_tiling_on_sc=False,
                                             skip_device_barrier=True),
        scratch_types=[
            pltpu.VMEM((t_rounds, g_sz), jnp.int32),
            pltpu.VMEM((g_sz, d), jnp.float32),
            pltpu.VMEM((g_sz, d), jnp.float32),
            pltpu.SemaphoreType.DMA,
            pltpu.SemaphoreType.DMA,
            pltpu.SemaphoreType.DMA,
            pltpu.SemaphoreType.DMA,
        ],
    )(table, idx3)


def _kmaj_idx(no, n, n_pad):
    idx = no[: n * 7].reshape(n, 7).T
    return jnp.pad(idx, ((0, 0), (0, n_pad - n))).reshape(-1)


def _gather7(table, no, n, n_pad):
    idx = jnp.pad(no[: n * 7], (0, 7 * (n_pad - n)))
    flat = _sc_gather(table, idx, 7 * n_pad)
    return flat.reshape(n_pad, 7 * table.shape[1])


def _tc_conv(gath, w, b):
    n_pad, kdim = gath.shape
    m = w.shape[0]
    rb = n_pad if n_pad <= 2688 else 2592
    grid = n_pad // rb

    def body(g_ref, w_ref, b_ref, o_ref):
        o_ref[...] = lax.dot_general(
            g_ref[...], w_ref[...], (((1,), (1,)), ((), ())),
            preferred_element_type=jnp.float32) + b_ref[...]

    return pl.pallas_call(
        body,
        grid=(grid,),
        in_specs=[
            pl.BlockSpec((rb, kdim), lambda i: (i, 0)),
            pl.BlockSpec((m, kdim), lambda i: (0, 0)),
            pl.BlockSpec((1, m), lambda i: (0, 0)),
        ],
        out_specs=pl.BlockSpec((rb, m), lambda i: (i, 0)),
        out_shape=jax.ShapeDtypeStruct((n_pad, m), jnp.float32),
    )(gath, w, b)


def _tc_bnrelu(h, g, be, n):
    n_pad, m = h.shape
    rcp = float(np.float32(1.0) / np.float32(n))
    wide = n == 10242
    rows = 128 if wide else 8

    def body(h_ref, g_ref, b_ref, o_ref):
        def tile_sum(transform):
            def step(i, acc):
                blk = transform(h_ref[pl.ds(i * rows, rows), :])
                rid = lax.broadcasted_iota(jnp.int32, (rows, m), 0) + i * rows
                return acc + jnp.where(rid < n, blk, 0.0)

            acc = lax.fori_loop(0, n_pad // rows, step,
                                jnp.zeros((rows, m), jnp.float32))
            if wide:
                p = acc[0:8]
                for j in range(1, 16):
                    p = p + acc[8 * j:8 * j + 8]
            else:
                p = acc
            q = p[0:4] + p[4:8]
            q = q[0:2] + q[2:4]
            return q[0:1] + q[1:2]

        mu = tile_sum(lambda b: b) * rcp
        var = tile_sum(lambda b: (b - mu) * (b - mu)) * rcp
        y = (h_ref[...] - mu) / jnp.sqrt(var + 1e-5) * g_ref[...] + b_ref[...]
        o_ref[...] = jnp.maximum(y, 0.0)

    return pl.pallas_call(
        body,
        out_shape=jax.ShapeDtypeStruct((n_pad, m), jnp.float32),
    )(h, g.reshape(1, m), be.reshape(1, m))


def _tc_pool(gath):
    _, num_pad, c = gath.shape

    def body(g_ref, mx_ref, mi_ref):
        best = g_ref[0]
        bi = jnp.zeros((num_pad, c), jnp.int32)
        for k in range(1, 7):
            cur = g_ref[k]
            upd = cur > best
            best = jnp.where(upd, cur, best)
            bi = jnp.where(upd, k, bi)
        mx_ref[...] = best
        mi_ref[...] = bi

    return pl.pallas_call(
        body,
        out_shape=(
            jax.ShapeDtypeStruct((num_pad, c), jnp.float32),
            jax.ShapeDtypeStruct((num_pad, c), jnp.int32),
        ),
    )(gath)


def _tc_unpool_vals(h, mi, num):
    num_pad, f = h.shape

    def body(h_ref, mi_ref, o_ref):
        mask = lax.broadcasted_iota(jnp.int32, (num_pad, f), 0) < num
        xv = jnp.where(mask, h_ref[...], 0.0)
        mv = mi_ref[...]
        for k in range(7):
            o_ref[k] = jnp.where(mv == k, xv, 0.0)

    return pl.pallas_call(
        body,
        out_shape=jax.ShapeDtypeStruct((7, num_pad, f), jnp.float32),
    )(h, mi)


def _sc_unpool(vals, nb4, ro):
    _, np_, f = vals.shape
    ngs = np_ // 32
    t_rounds = -(-ngs // NT)
    rows_pt = ro // NT
    zeros = jnp.zeros((ro, f), jnp.float32)

    mesh = plsc.VectorSubcoreMesh(core_axis_name="c", subcore_axis_name="s")

    def body(vals_hbm, nb_hbm, z_hbm, out_hbm, val_v, nb_v, shared):
        cid = lax.axis_index("c")
        sid = lax.axis_index("s")

        @pl.when(cid == 0)
        def _():
            pltpu.sync_copy(z_hbm.at[pl.ds(sid * rows_pt, rows_pt)],
                            shared.at[pl.ds(sid * rows_pt, rows_pt)])

        plsc.subcore_barrier()

        @pl.when(cid == 0)
        def _():
            for t in range(t_rounds):
                grp = t * NT + sid

                @pl.when(grp < ngs)
                def _():
                    pltpu.sync_copy(nb_hbm.at[grp], nb_v)
                    for k in range(7):
                        pltpu.sync_copy(vals_hbm.at[k, pl.ds(grp * 32, 32)], val_v)
                        pltpu.sync_copy(val_v, shared.at[nb_v.at[k]], add=True)

        plsc.subcore_barrier()

        @pl.when(cid == 0)
        def _():
            pltpu.sync_copy(shared.at[pl.ds(sid * rows_pt, rows_pt)],
                            out_hbm.at[pl.ds(sid * rows_pt, rows_pt)])

    return pl.kernel(
        body,
        out_type=jax.ShapeDtypeStruct((ro, f), jnp.float32),
        mesh=mesh,
        compiler_params=pltpu.CompilerParams(use_tc_tiling_on_sc=False,
                                             skip_device_barrier=True),
        scratch_types=[
            pltpu.VMEM((32, f), jnp.float32),
            pltpu.VMEM((7, 32), jnp.int32),
            pltpu.VMEM_SHARED((ro, f), jnp.float32),
        ],
    )(vals, nb4, zeros)


def _unpool(h, mi, no, num, ro):
    vals = _tc_unpool_vals(h, mi, num)
    np_ = h.shape[0]
    nb = no[: num * 7].reshape(num, 7).T
    nb = jnp.pad(nb, ((0, 0), (0, np_ - num)))
    nb4 = nb.reshape(7, np_ // 32, 32).transpose(1, 0, 2)
    return _sc_unpool(vals, nb4, ro)


def kernel(x, params, neigh_orders):
    no = neigh_orders
    npad = {n: _rup(n, 128) for n in _LEVELS}

    def cbr(h, name, n):
        cin = params["W_" + name].shape[1] // 7
        cout = params["W_" + name].shape[0]
        gath = _gather7(h, no[str(n)], n, npad[n])
        if cin != h.shape[1]:
            gath = gath.reshape(npad[n], 7, h.shape[1])[:, :, :cin]
            gath = gath.reshape(npad[n], 7 * cin)
        hh = _tc_conv(gath, params["W_" + name],
                      params["b_" + name].reshape(1, cout))
        return _tc_bnrelu(hh, params["g_" + name], params["be_" + name], n)

    def pool(h, n):
        num = (n + 6) // 4
        idx = _kmaj_idx(no[str(n)], num, npad[num])
        gath = _sc_gather(h, idx, 7 * npad[num]).reshape(7, npad[num], h.shape[1])
        return _tc_pool(gath)

    h = jnp.pad(x, ((0, 0), (0, 13)))
    h = cbr(h, "1_1", 10242)
    h = cbr(h, "1_2", 10242)
    h, mi1 = pool(h, 10242)
    h = cbr(h, "2_1", 2562)
    h = cbr(h, "2_2", 2562)
    h, mi2 = pool(h, 2562)
    h = cbr(h, "3_1", 642)
    h = cbr(h, "3_2", 642)
    h, mi3 = pool(h, 642)
    h = cbr(h, "4_1", 162)
    h = cbr(h, "4_2", 162)
    h, mi4 = pool(h, 162)
    h = cbr(h, "5_1", 42)
    h = cbr(h, "5_2", 42)
    h = _unpool(h, mi4, no["162"], 42, _rup(162, 16 * 32))
    h = cbr(h, "6_1", 162)
    h = cbr(h, "6_2", 162)
    h = _unpool(h, mi3, no["642"], 162, _rup(642, 16 * 32))
    h = cbr(h, "7_1", 642)
    h = cbr(h, "7_2", 642)
    h = _unpool(h, mi2, no["2562"], 642, _rup(2562, 16 * 32))
    h = cbr(h, "8_1", 2562)
    h = cbr(h, "8_2", 2562)
    h = _unpool(h, mi1, no["10242"], 2562, _rup(10242, 16 * 32))
    h = cbr(h, "9_1", 10242)
    h = cbr(h, "9_2", 10242)
    w10 = jnp.pad(params["W_10"], ((0, 128 - 36), (0, 0)))
    gath = _gather7(h, no["10242"], 10242, npad[10242])
    b10 = jnp.pad(params["b_10"], (0, 128 - 36)).reshape(1, 128)
    out = _tc_conv(gath, w10, b10)
    return out[:10242, :36]
